# baseline (speedup 1.0000x reference)
import functools

import jax
import jax.numpy as jnp
from jax import lax
from jax.experimental import pallas as pl
from jax.experimental.pallas import tpu as pltpu
from jax.experimental.pallas import tpu_sc as plsc

N = 320000
H = 128
G = 1024

_BLK = 4096
_BLK_T = 512

_SEG_SCS = (104, 104, 72, 32)
_SEG_LENS = tuple(s * 1024 for s in _SEG_SCS)
_TAIL_LEN = N - sum(_SEG_LENS)


def _gate_body(emb_ref, feat_ref, wg_ref, bg_ref, wt_ref, bt_ref, out_ref):
    emb = emb_ref[...]
    feat = feat_ref[...]
    wg = wg_ref[...]
    cdims = (((1,), (1,)), ((), ()))
    z = (lax.dot_general(emb, wg[:, :H], cdims,
                         preferred_element_type=jnp.float32)
         + lax.dot_general(feat, wg[:, H:], cdims,
                           preferred_element_type=jnp.float32)
         + bg_ref[...])
    gate = jax.nn.sigmoid(z)
    t = jnp.tanh(
        lax.dot_general(emb, wt_ref[...], cdims,
                        preferred_element_type=jnp.float32)
        + bt_ref[...])
    out_ref[...] = gate * t


def _gate_call(emb, feat, wg, bg2, wt, bt2, row0, seg_len, blk):
    nblocks = seg_len // blk
    blk0 = row0 // blk
    row_spec = pl.BlockSpec((blk, H), lambda i: (i + blk0, 0))
    out_spec = pl.BlockSpec((blk, H), lambda i: (i, 0))
    wg_spec = pl.BlockSpec((H, 2 * H), lambda i: (0, 0))
    wt_spec = pl.BlockSpec((H, H), lambda i: (0, 0))
    b_spec = pl.BlockSpec((1, H), lambda i: (0, 0))
    return pl.pallas_call(
        _gate_body,
        grid=(nblocks,),
        in_specs=[row_spec, row_spec, wg_spec, b_spec, wt_spec, b_spec],
        out_specs=out_spec,
        out_shape=jax.ShapeDtypeStruct((seg_len, H), jnp.float32),
    )(emb, feat, wg, bg2, wt, bt2)


_RP = 2504
_SC_FULL = N // 1024
_GPS = G // 16
_Q = 256


def _make_segsum_body(sc0, scnt, has_tail):
    q, r = divmod(scnt * 4, 32)

    def body(gated, *rest):
        if has_tail:
            gated_tail, ids, zeros64, out, idx_v, rows0, rows1, stage_v, \
                acc, sem0, sem1 = rest
        else:
            ids, zeros64, out, idx_v, rows0, rows1, stage_v, \
                acc, sem0, sem1 = rest
        c = lax.axis_index("c")
        s = lax.axis_index("s")
        wid = s * 2 + c
        bufs = (rows0, rows1)
        sems = (sem0, sem1)

        pltpu.sync_copy(zeros64, stage_v)
        pltpu.sync_copy(stage_v, acc.at[pl.ds(s * _GPS, _GPS)])
        plsc.subcore_barrier()

        qbase = wid * q + jnp.minimum(wid, r)
        qcnt = q + (wid < r)

        def gather_start(node0, buf, sem):
            pltpu.make_async_copy(gated.at[pl.ds(node0, _Q)], buf,
                                  sem).start()

        def gather_wait(buf, sem):
            pltpu.make_async_copy(gated.at[pl.ds(0, _Q)], buf, sem).wait()

        gather_start(qbase * _Q, rows0, sem0)

        def pair_body(p, carry):
            for b in range(2):
                j = 2 * p + b

                @pl.when(j < qcnt)
                def _step():
                    qtr = qbase + j
                    sc = qtr // 4
                    qq = qtr % 4

                    @pl.when((qq == 0) | (j == 0))
                    def _load_idx():
                        pltpu.sync_copy(ids.at[pl.ds((sc0 + sc) * 8, 8)],
                                        idx_v)

                    gather_wait(bufs[b], sems[b])

                    @pl.when(j + 1 < qcnt)
                    def _prefetch_next():
                        gather_start((qbase + j + 1) * _Q, bufs[1 - b],
                                     sems[1 - b])

                    for h in range(2):
                        pltpu.sync_copy(bufs[b].at[pl.ds(h * H, H)],
                                        acc.at[idx_v.at[2 * qq + h]],
                                        add=True)
            return carry

        lax.fori_loop(0, (qcnt + 1) // 2, pair_body, 0)

        if has_tail:
            @pl.when(wid == 31)
            def _tail():
                pltpu.sync_copy(ids.at[pl.ds(_SC_FULL * 8, 8)], idx_v)
                for qq in range(2):
                    pltpu.sync_copy(gated_tail.at[pl.ds(qq * _Q, _Q)],
                                    rows0)
                    for h in range(2):
                        pltpu.sync_copy(rows0.at[pl.ds(h * H, H)],
                                        acc.at[idx_v.at[2 * qq + h]],
                                        add=True)

        plsc.subcore_barrier()

        pltpu.sync_copy(acc.at[pl.ds(s * _GPS, _GPS)], stage_v)
        pltpu.sync_copy(stage_v, out.at[c].at[pl.ds(s * _GPS, _GPS)])

    return body


def _make_segsum_call(sc0, scnt, has_tail):
    return functools.partial(
        pl.kernel,
        out_type=jax.ShapeDtypeStruct((2, G, H), jnp.float32),
        mesh=plsc.VectorSubcoreMesh(core_axis_name="c", subcore_axis_name="s"),
        scratch_types=[
            pltpu.VMEM((8, H), jnp.int32),
            pltpu.VMEM((_Q, H), jnp.float32),
            pltpu.VMEM((_Q, H), jnp.float32),
            pltpu.VMEM((_GPS, H), jnp.float32),
            pltpu.VMEM_SHARED((G, H), jnp.float32),
            pltpu.SemaphoreType.DMA,
            pltpu.SemaphoreType.DMA,
        ],
    )(_make_segsum_body(sc0, scnt, has_tail))


_NSEG = len(_SEG_SCS)
_SEG_SC0S = tuple(sum(_SEG_SCS[:k]) for k in range(_NSEG))
_segsum_calls = tuple(
    _make_segsum_call(_SEG_SC0S[k], _SEG_SCS[k], k == _NSEG - 1)
    for k in range(_NSEG)
)


def _combine_body(*refs):
    p_refs = refs[:_NSEG]
    wo_ref, bo_ref, pred_ref, repr_ref = refs[_NSEG:]
    grepr = p_refs[0][0] + p_refs[0][1]
    for p in p_refs[1:]:
        grepr = grepr + p[0] + p[1]
    repr_ref[...] = grepr
    pred_ref[...] = (jnp.sum(grepr * wo_ref[...], axis=1, keepdims=True)
                     + bo_ref[...])


def _combine_call(partials, wo, bo2):
    return pl.pallas_call(
        _combine_body,
        out_shape=(
            jax.ShapeDtypeStruct((G, 1), jnp.float32),
            jax.ShapeDtypeStruct((G, H), jnp.float32),
        ),
    )(*partials, wo, bo2)


def kernel(node_embeddings, initial_features, graph_nodes_list, num_graphs,
           Wg, bg, Wt, bt, Wo, bo):
    bg2 = bg.reshape(1, H)
    bt2 = bt.reshape(1, H)
    ids2d = jnp.concatenate(
        [graph_nodes_list,
         jnp.zeros((_RP * H - N,), jnp.int32)]).reshape(_RP, H)
    zeros64 = jnp.zeros((_GPS, H), jnp.float32)

    row0s = tuple(s * 1024 for s in _SEG_SC0S)
    gated = [
        _gate_call(node_embeddings, initial_features, Wg, bg2, Wt, bt2,
                   row0s[k], _SEG_LENS[k], _BLK)
        for k in range(_NSEG)
    ]
    gated_t = _gate_call(node_embeddings, initial_features, Wg, bg2, Wt, bt2,
                         sum(_SEG_LENS), _TAIL_LEN, _BLK_T)
    partials = [
        _segsum_calls[k](*((gated[k], gated_t) if k == _NSEG - 1
                           else (gated[k],)), ids2d, zeros64)
        for k in range(_NSEG)
    ]
    pred, graph_repr = _combine_call(partials, Wo, bo.reshape(1, 1))
    return pred.reshape(G), graph_repr


# speedup vs baseline: 1.0117x; 1.0117x over previous
import functools

import jax
import jax.numpy as jnp
from jax import lax
from jax.experimental import pallas as pl
from jax.experimental.pallas import tpu as pltpu
from jax.experimental.pallas import tpu_sc as plsc

N = 320000
H = 128
G = 1024

_BLK = 4096
_BLK_T = 512

_SEG_SCS = (104, 104, 104)
_SEG_LENS = tuple(s * 1024 for s in _SEG_SCS)
_TAIL_LEN = N - sum(_SEG_LENS)


def _gate_body(emb_ref, feat_ref, wg_ref, bg_ref, wt_ref, bt_ref, out_ref):
    emb = emb_ref[...]
    feat = feat_ref[...]
    wg = wg_ref[...]
    cdims = (((1,), (1,)), ((), ()))
    z = (lax.dot_general(emb, wg[:, :H], cdims,
                         preferred_element_type=jnp.float32)
         + lax.dot_general(feat, wg[:, H:], cdims,
                           preferred_element_type=jnp.float32)
         + bg_ref[...])
    gate = jax.nn.sigmoid(z)
    t = jnp.tanh(
        lax.dot_general(emb, wt_ref[...], cdims,
                        preferred_element_type=jnp.float32)
        + bt_ref[...])
    out_ref[...] = gate * t


def _gate_call(emb, feat, wg, bg2, wt, bt2, row0, seg_len, blk):
    nblocks = seg_len // blk
    blk0 = row0 // blk
    row_spec = pl.BlockSpec((blk, H), lambda i: (i + blk0, 0))
    out_spec = pl.BlockSpec((blk, H), lambda i: (i, 0))
    wg_spec = pl.BlockSpec((H, 2 * H), lambda i: (0, 0))
    wt_spec = pl.BlockSpec((H, H), lambda i: (0, 0))
    b_spec = pl.BlockSpec((1, H), lambda i: (0, 0))
    return pl.pallas_call(
        _gate_body,
        grid=(nblocks,),
        in_specs=[row_spec, row_spec, wg_spec, b_spec, wt_spec, b_spec],
        out_specs=out_spec,
        out_shape=jax.ShapeDtypeStruct((seg_len, H), jnp.float32),
    )(emb, feat, wg, bg2, wt, bt2)


_RP = 2504
_SC_FULL = N // 1024
_GPS = G // 16
_Q = 256


def _make_segsum_body(sc0, scnt, has_tail):
    q, r = divmod(scnt * 4, 32)

    def body(gated, *rest):
        if has_tail:
            gated_tail, ids, zeros64, out, idx_v, rows0, rows1, \
                acc, sem0, sem1 = rest
        else:
            ids, zeros64, out, idx_v, rows0, rows1, \
                acc, sem0, sem1 = rest
        c = lax.axis_index("c")
        s = lax.axis_index("s")
        wid = s * 2 + c
        bufs = (rows0, rows1)
        sems = (sem0, sem1)

        pltpu.sync_copy(zeros64, acc.at[pl.ds(s * _GPS, _GPS)])
        plsc.subcore_barrier()

        qbase = wid * q + jnp.minimum(wid, r)
        qcnt = q + (wid < r)

        def gather_start(node0, buf, sem):
            pltpu.make_async_copy(gated.at[pl.ds(node0, _Q)], buf,
                                  sem).start()

        def gather_wait(buf, sem):
            pltpu.make_async_copy(gated.at[pl.ds(0, _Q)], buf, sem).wait()

        gather_start(qbase * _Q, rows0, sem0)

        def pair_body(p, carry):
            for b in range(2):
                j = 2 * p + b

                @pl.when(j < qcnt)
                def _step():
                    qtr = qbase + j
                    sc = qtr // 4
                    qq = qtr % 4

                    @pl.when((qq == 0) | (j == 0))
                    def _load_idx():
                        pltpu.sync_copy(ids.at[pl.ds((sc0 + sc) * 8, 8)],
                                        idx_v)

                    gather_wait(bufs[b], sems[b])

                    @pl.when(j + 1 < qcnt)
                    def _prefetch_next():
                        gather_start((qbase + j + 1) * _Q, bufs[1 - b],
                                     sems[1 - b])

                    for h in range(2):
                        pltpu.sync_copy(bufs[b].at[pl.ds(h * H, H)],
                                        acc.at[idx_v.at[2 * qq + h]],
                                        add=True)
            return carry

        lax.fori_loop(0, (qcnt + 1) // 2, pair_body, 0)

        if has_tail:
            @pl.when(wid == 31)
            def _tail():
                pltpu.sync_copy(ids.at[pl.ds(_SC_FULL * 8, 8)], idx_v)
                for qq in range(2):
                    pltpu.sync_copy(gated_tail.at[pl.ds(qq * _Q, _Q)],
                                    rows0)
                    for h in range(2):
                        pltpu.sync_copy(rows0.at[pl.ds(h * H, H)],
                                        acc.at[idx_v.at[2 * qq + h]],
                                        add=True)

        plsc.subcore_barrier()

        pltpu.sync_copy(acc.at[pl.ds(s * _GPS, _GPS)],
                        out.at[c].at[pl.ds(s * _GPS, _GPS)])

    return body


def _make_segsum_call(sc0, scnt, has_tail):
    return functools.partial(
        pl.kernel,
        out_type=jax.ShapeDtypeStruct((2, G, H), jnp.float32),
        mesh=plsc.VectorSubcoreMesh(core_axis_name="c", subcore_axis_name="s"),
        scratch_types=[
            pltpu.VMEM((8, H), jnp.int32),
            pltpu.VMEM((_Q, H), jnp.float32),
            pltpu.VMEM((_Q, H), jnp.float32),
            pltpu.VMEM_SHARED((G, H), jnp.float32),
            pltpu.SemaphoreType.DMA,
            pltpu.SemaphoreType.DMA,
        ],
    )(_make_segsum_body(sc0, scnt, has_tail))


_NSEG = len(_SEG_SCS)
_SEG_SC0S = tuple(sum(_SEG_SCS[:k]) for k in range(_NSEG))
_segsum_calls = tuple(
    _make_segsum_call(_SEG_SC0S[k], _SEG_SCS[k], k == _NSEG - 1)
    for k in range(_NSEG)
)


def _combine_body(*refs):
    p_refs = refs[:_NSEG]
    wo_ref, bo_ref, pred_ref, repr_ref = refs[_NSEG:]
    grepr = p_refs[0][0] + p_refs[0][1]
    for p in p_refs[1:]:
        grepr = grepr + p[0] + p[1]
    repr_ref[...] = grepr
    pred_ref[...] = (jnp.sum(grepr * wo_ref[...], axis=1, keepdims=True)
                     + bo_ref[...])


def _combine_call(partials, wo, bo2):
    return pl.pallas_call(
        _combine_body,
        out_shape=(
            jax.ShapeDtypeStruct((G, 1), jnp.float32),
            jax.ShapeDtypeStruct((G, H), jnp.float32),
        ),
    )(*partials, wo, bo2)


def kernel(node_embeddings, initial_features, graph_nodes_list, num_graphs,
           Wg, bg, Wt, bt, Wo, bo):
    bg2 = bg.reshape(1, H)
    bt2 = bt.reshape(1, H)
    ids2d = jnp.concatenate(
        [graph_nodes_list,
         jnp.zeros((_RP * H - N,), jnp.int32)]).reshape(_RP, H)
    zeros64 = jnp.zeros((_GPS, H), jnp.float32)

    row0s = tuple(s * 1024 for s in _SEG_SC0S)
    gated = [
        _gate_call(node_embeddings, initial_features, Wg, bg2, Wt, bt2,
                   row0s[k], _SEG_LENS[k], _BLK)
        for k in range(_NSEG)
    ]
    gated_t = _gate_call(node_embeddings, initial_features, Wg, bg2, Wt, bt2,
                         sum(_SEG_LENS), _TAIL_LEN, _BLK_T)
    partials = [
        _segsum_calls[k](*((gated[k], gated_t) if k == _NSEG - 1
                           else (gated[k],)), ids2d, zeros64)
        for k in range(_NSEG)
    ]
    pred, graph_repr = _combine_call(partials, Wo, bo.reshape(1, 1))
    return pred.reshape(G), graph_repr
